# Initial kernel scaffold; baseline (speedup 1.0000x reference)
import jax
import jax.numpy as jnp
from jax.experimental import pallas as pl


def kernel(hours, days, months, hour_table, day_table, month_table):
    raise NotImplementedError("write your pallas kernel here")


# speedup vs baseline: 16.6697x; 16.6697x over previous
import functools

import jax
import jax.numpy as jnp
from jax import lax
from jax.experimental import pallas as pl
from jax.experimental.pallas import tpu as pltpu
from jax.experimental.pallas import tpu_sc as plsc

B, S, D = 16384, 200, 32
N = B * S

NH, ND, NM = 24, 7, 12
CT_ROWS = NH * ND * NM
TCAT = 48

_info = plsc.get_sparse_core_info()
NC, NS, L = _info.num_cores, _info.num_subcores, _info.num_lanes
NW = NC * NS
PER_W = N // NW
CHUNK = 2048
NCH = PER_W // CHUNK
GATHER = 128
NG = CHUNK // GATHER


def _ct_body(tcat_ref, ct_ref):
    rr = lax.broadcasted_iota(jnp.int32, (CT_ROWS, TCAT), 0)
    cc = lax.broadcasted_iota(jnp.int32, (CT_ROWS, TCAT), 1)
    h = rr // (ND * NM)
    rem = rr - h * (ND * NM)
    d = rem // NM
    m = rem - d * NM
    mh = ((cc == h) | (cc == NH + d) | (cc == NH + ND + m)).astype(jnp.float32)
    ct_ref[...] = lax.dot_general(
        mh, tcat_ref[...], (((1,), (0,)), ((), ())),
        preferred_element_type=jnp.float32,
        precision=lax.Precision.HIGHEST)


def _build_combined_table(hour_table, day_table, month_table):
    tcat = jnp.concatenate(
        [hour_table, day_table, month_table,
         jnp.zeros((TCAT - NH - ND - NM, D), jnp.float32)], axis=0)
    return pl.pallas_call(
        _ct_body,
        out_shape=jax.ShapeDtypeStruct((CT_ROWS, D), jnp.float32),
    )(tcat)


def _sc_body(ct_hbm, h_hbm, d_hbm, m_hbm, out_hbm,
             h_v, d_v, m_v, cidx_v, rows_v, sem_g):
    wid = lax.axis_index("s") * NC + lax.axis_index("c")
    base = wid * PER_W

    def chunk_body(ci, carry):
        off = base + ci * CHUNK
        pltpu.sync_copy(h_hbm.at[pl.ds(off, CHUNK)], h_v)
        pltpu.sync_copy(d_hbm.at[pl.ds(off, CHUNK)], d_v)
        pltpu.sync_copy(m_hbm.at[pl.ds(off, CHUNK)], m_v)

        def vec_body(j, c2):
            p = j * L
            hv = h_v[pl.ds(p, L)]
            dv = d_v[pl.ds(p, L)]
            mv = m_v[pl.ds(p, L)]
            g = j // (GATHER // L)
            col = (j - g * (GATHER // L)) * L
            cidx_v[g, pl.ds(col, L)] = hv * (ND * NM) + dv * NM + mv
            return c2

        lax.fori_loop(0, CHUNK // L, vec_body, 0, unroll=8)

        copies = []
        for g in range(NG):
            copies.append(pltpu.async_copy(
                ct_hbm.at[cidx_v.at[g]],
                rows_v.at[pl.ds(g * GATHER, GATHER)], sem_g))
        for c in copies:
            c.wait()

        pltpu.sync_copy(rows_v, out_hbm.at[pl.ds(off, CHUNK)])
        return carry

    lax.fori_loop(0, NCH, chunk_body, 0)


_sc_call = functools.partial(
    pl.kernel,
    out_type=jax.ShapeDtypeStruct((N, D), jnp.float32),
    mesh=plsc.VectorSubcoreMesh(core_axis_name="c", subcore_axis_name="s"),
    scratch_types=[
        pltpu.VMEM((CHUNK,), jnp.int32),
        pltpu.VMEM((CHUNK,), jnp.int32),
        pltpu.VMEM((CHUNK,), jnp.int32),
        pltpu.VMEM((NG, GATHER), jnp.int32),
        pltpu.VMEM((CHUNK, D), jnp.float32),
        pltpu.SemaphoreType.DMA,
    ],
    compiler_params=pltpu.CompilerParams(use_tc_tiling_on_sc=False),
)(_sc_body)


def kernel(hours, days, months, hour_table, day_table, month_table):
    h = hours.astype(jnp.int32).reshape(N)
    d = days.astype(jnp.int32).reshape(N)
    m = months.astype(jnp.int32).reshape(N)
    ct = _build_combined_table(hour_table, day_table, month_table)
    out = _sc_call(ct, h, d, m)
    return out.reshape(B, S, D)
